# baseline (speedup 1.0000x reference)
import jax
import jax.numpy as jnp
from jax.experimental import pallas as pl
from jax.experimental.pallas import tpu as pltpu

NUM_EXPERTS = 8
TOP_K = 2
BLK = 1024


def _router_body(x_ref, wt_ref, b_ref, logits_ref, probs_ref, w_ref, idx_ref):
    x = x_ref[...]
    wt = wt_ref[...]
    logits = jnp.dot(x, wt, preferred_element_type=jnp.float32) + b_ref[...]

    lt = logits.T
    logits_ref[...] = lt
    m = jnp.max(lt, axis=0, keepdims=True)
    e = jnp.exp(lt - m)
    s = jnp.sum(e, axis=0, keepdims=True)
    pt = e / s
    probs_ref[...] = pt

    eids = jax.lax.broadcasted_iota(jnp.int32, pt.shape, 0)
    p1 = jnp.max(pt, axis=0, keepdims=True)
    i1 = jnp.min(jnp.where(pt == p1, eids, NUM_EXPERTS), axis=0,
                 keepdims=True)
    pt2 = jnp.where(eids == i1, -1.0, pt)
    p2 = jnp.max(pt2, axis=0, keepdims=True)
    i2 = jnp.min(jnp.where(pt2 == p2, eids, NUM_EXPERTS), axis=0,
                 keepdims=True)

    t = jnp.exp(p2 - p1)
    denom = 1.0 + t
    w_ref[...] = jnp.concatenate([1.0 / denom, t / denom], axis=0)
    idx_ref[...] = jnp.concatenate([i1, i2], axis=0)


@jax.jit
def kernel(inputs, W, b):
    B, S, D = inputs.shape
    N = B * S
    x2d = inputs.reshape(N, D)
    wt = W.T
    b2d = b.reshape(1, NUM_EXPERTS)

    grid = (N // BLK,)
    logits_t, probs_t, w_t, idx_t = pl.pallas_call(
        _router_body,
        grid=grid,
        in_specs=[
            pl.BlockSpec((BLK, D), lambda i: (i, 0)),
            pl.BlockSpec((D, NUM_EXPERTS), lambda i: (0, 0)),
            pl.BlockSpec((1, NUM_EXPERTS), lambda i: (0, 0)),
        ],
        out_specs=[
            pl.BlockSpec((NUM_EXPERTS, BLK), lambda i: (0, i)),
            pl.BlockSpec((NUM_EXPERTS, BLK), lambda i: (0, i)),
            pl.BlockSpec((TOP_K, BLK), lambda i: (0, i)),
            pl.BlockSpec((TOP_K, BLK), lambda i: (0, i)),
        ],
        out_shape=[
            jax.ShapeDtypeStruct((NUM_EXPERTS, N), jnp.float32),
            jax.ShapeDtypeStruct((NUM_EXPERTS, N), jnp.float32),
            jax.ShapeDtypeStruct((TOP_K, N), jnp.float32),
            jax.ShapeDtypeStruct((TOP_K, N), jnp.int32),
        ],
    )(x2d, wt, b2d)

    return (
        logits_t.T.reshape(B, S, NUM_EXPERTS),
        probs_t.T.reshape(B, S, NUM_EXPERTS),
        w_t.T.reshape(B, S, TOP_K),
        idx_t.T.reshape(B, S, TOP_K),
    )


# speedup vs baseline: 1.2062x; 1.2062x over previous
import jax
import jax.numpy as jnp
from jax.experimental import pallas as pl
from jax.experimental.pallas import tpu as pltpu

NUM_EXPERTS = 8
TOP_K = 2
BLK = 2048


def _router_body(x_ref, wt_ref, b_ref, logits_ref, probs_ref, w_ref, idx_ref):
    x = x_ref[...]
    wt = wt_ref[...]
    logits = jnp.dot(x, wt, preferred_element_type=jnp.float32) + b_ref[...]

    lt = logits.T
    logits_ref[...] = lt
    m = jnp.max(lt, axis=0, keepdims=True)
    e = jnp.exp(lt - m)
    s = jnp.sum(e, axis=0, keepdims=True)
    pt = e / s
    probs_ref[...] = pt

    eids = jax.lax.broadcasted_iota(jnp.int32, pt.shape, 0)
    p1 = jnp.max(pt, axis=0, keepdims=True)
    i1 = jnp.min(jnp.where(pt == p1, eids, NUM_EXPERTS), axis=0,
                 keepdims=True)
    pt2 = jnp.where(eids == i1, -1.0, pt)
    p2 = jnp.max(pt2, axis=0, keepdims=True)
    i2 = jnp.min(jnp.where(pt2 == p2, eids, NUM_EXPERTS), axis=0,
                 keepdims=True)

    t = jnp.exp(p2 - p1)
    denom = 1.0 + t
    w_ref[...] = jnp.concatenate([1.0 / denom, t / denom], axis=0)
    idx_ref[...] = jnp.concatenate([i1, i2], axis=0)


@jax.jit
def kernel(inputs, W, b):
    B, S, D = inputs.shape
    N = B * S
    x2d = inputs.reshape(N, D)
    wt = W.T
    b2d = b.reshape(1, NUM_EXPERTS)

    grid = (N // BLK,)
    logits_t, probs_t, w_t, idx_t = pl.pallas_call(
        _router_body,
        grid=grid,
        in_specs=[
            pl.BlockSpec((BLK, D), lambda i: (i, 0)),
            pl.BlockSpec((D, NUM_EXPERTS), lambda i: (0, 0)),
            pl.BlockSpec((1, NUM_EXPERTS), lambda i: (0, 0)),
        ],
        out_specs=[
            pl.BlockSpec((NUM_EXPERTS, BLK), lambda i: (0, i)),
            pl.BlockSpec((NUM_EXPERTS, BLK), lambda i: (0, i)),
            pl.BlockSpec((TOP_K, BLK), lambda i: (0, i)),
            pl.BlockSpec((TOP_K, BLK), lambda i: (0, i)),
        ],
        out_shape=[
            jax.ShapeDtypeStruct((NUM_EXPERTS, N), jnp.float32),
            jax.ShapeDtypeStruct((NUM_EXPERTS, N), jnp.float32),
            jax.ShapeDtypeStruct((TOP_K, N), jnp.float32),
            jax.ShapeDtypeStruct((TOP_K, N), jnp.int32),
        ],
    )(x2d, wt, b2d)

    return (
        logits_t.T.reshape(B, S, NUM_EXPERTS),
        probs_t.T.reshape(B, S, NUM_EXPERTS),
        w_t.T.reshape(B, S, TOP_K),
        idx_t.T.reshape(B, S, TOP_K),
    )
